# initial kernel scaffold (unmeasured)
import jax
import jax.numpy as jnp
from jax import lax
from jax.experimental import pallas as pl
from jax.experimental.pallas import tpu as pltpu

N_DEV = 4
SQ = 1024
SKV = 1024
HQ_LOC = 8
DH = 128
DM = 1024
DLOC = HQ_LOC * DH
CHUNK = SQ // N_DEV
SCALE = 0.08838834764831843
N_HOPS = 2 * (N_DEV - 1)


def _body(x_ref, wq_ref, k_ref, v_ref, wo_ref, out_ref,
          partial_ref, comm_ref, send_sems, recv_sems):
    my = lax.axis_index("i")
    left = (my + N_DEV - 1) % N_DEV
    right = (my + 1) % N_DEV

    xm = x_ref[0]
    q_all = jnp.dot(xm, wq_ref[...], preferred_element_type=jnp.float32)

    ib = lax.broadcasted_iota(jnp.int32, (SQ, SKV), 0) // 64 % 4
    jb = lax.broadcasted_iota(jnp.int32, (SQ, SKV), 1) // 64 % 4
    mask = ib == jb

    ctxs = []
    for h in range(HQ_LOC):
        q = q_all[:, h * DH:(h + 1) * DH]
        k = k_ref[0, :, h, :]
        v = v_ref[0, :, h, :]
        s = jnp.dot(q, k.T, preferred_element_type=jnp.float32) * SCALE
        s = jnp.where(mask, s, -1e9)
        m = jnp.max(s, axis=-1, keepdims=True)
        w = jnp.exp(s - m)
        w = w / jnp.sum(w, axis=-1, keepdims=True)
        ctxs.append(jnp.dot(w, v, preferred_element_type=jnp.float32))
    ctx = jnp.concatenate(ctxs, axis=1)
    partial_ref[...] = jnp.dot(ctx, wo_ref[...],
                               preferred_element_type=jnp.float32)

    barrier_sem = pltpu.get_barrier_semaphore()
    for nbr in (left, right):
        pl.semaphore_signal(barrier_sem, inc=1, device_id=(nbr,),
                            device_id_type=pl.DeviceIdType.MESH)
    pl.semaphore_wait(barrier_sem, 2)

    for s in range(N_DEV - 1):
        cs = (my + N_DEV - s) % N_DEV
        rdma = pltpu.make_async_remote_copy(
            src_ref=partial_ref.at[pl.ds(cs * CHUNK, CHUNK), :],
            dst_ref=comm_ref.at[s],
            send_sem=send_sems.at[s],
            recv_sem=recv_sems.at[s],
            device_id=(right,),
            device_id_type=pl.DeviceIdType.MESH,
        )
        rdma.start()
        rdma.wait()
        cr = (my + N_DEV - 1 - s) % N_DEV
        partial_ref[pl.ds(cr * CHUNK, CHUNK), :] = (
            partial_ref[pl.ds(cr * CHUNK, CHUNK), :] + comm_ref[s]
        )

    own = (my + 1) % N_DEV
    out_ref[0, pl.ds(own * CHUNK, CHUNK), :] = partial_ref[
        pl.ds(own * CHUNK, CHUNK), :]

    for g in range(N_DEV - 1):
        s = (N_DEV - 1) + g
        if g == 0:
            src = partial_ref.at[pl.ds(own * CHUNK, CHUNK), :]
        else:
            src = comm_ref.at[s - 1]
        rdma = pltpu.make_async_remote_copy(
            src_ref=src,
            dst_ref=comm_ref.at[s],
            send_sem=send_sems.at[s],
            recv_sem=recv_sems.at[s],
            device_id=(right,),
            device_id_type=pl.DeviceIdType.MESH,
        )
        rdma.start()
        rdma.wait()
        cr = (my + N_DEV - g) % N_DEV
        out_ref[0, pl.ds(cr * CHUNK, CHUNK), :] = comm_ref[s]


def kernel(x, Wq, K_ext, V_ext, Wo):
    i = lax.axis_index("i")
    wq_loc = lax.dynamic_slice(Wq, (0, i * DLOC), (DM, DLOC))
    wo_loc = lax.dynamic_slice(Wo, (i * DLOC, 0), (DLOC, DM))

    return pl.pallas_call(
        _body,
        out_shape=jax.ShapeDtypeStruct((1, SQ, DM), jnp.float32),
        in_specs=[pl.BlockSpec(memory_space=pltpu.VMEM)] * 5,
        out_specs=pl.BlockSpec(memory_space=pltpu.VMEM),
        scratch_shapes=[
            pltpu.VMEM((SQ, DM), jnp.float32),
            pltpu.VMEM((N_HOPS, CHUNK, DM), jnp.float32),
            pltpu.SemaphoreType.DMA((N_HOPS,)),
            pltpu.SemaphoreType.DMA((N_HOPS,)),
        ],
        compiler_params=pltpu.CompilerParams(collective_id=0),
    )(x, wq_loc, K_ext, V_ext, Wo_loc := wo_loc)


# baseline (device time: 116431 ns/iter reference)
import jax
import jax.numpy as jnp
from jax import lax
from jax.experimental import pallas as pl
from jax.experimental.pallas import tpu as pltpu

N_DEV = 4
SQ = 1024
SKV = 1024
HQ_LOC = 8
DH = 128
DM = 1024
DLOC = HQ_LOC * DH
CHUNK = SQ // N_DEV
SCALE = 0.08838834764831843
N_HOPS = 2 * (N_DEV - 1)


def _body(x_ref, wq_ref, k_ref, v_ref, wo_ref, out_ref,
          partial_ref, comm_ref, send_sems, recv_sems):
    my = lax.axis_index("i")
    left = (my + N_DEV - 1) % N_DEV
    right = (my + 1) % N_DEV

    xm = x_ref[0]
    q_all = jnp.dot(xm, wq_ref[...], preferred_element_type=jnp.float32)

    ib = lax.broadcasted_iota(jnp.int32, (SQ, SKV), 0) // 64 % 4
    jb = lax.broadcasted_iota(jnp.int32, (SQ, SKV), 1) // 64 % 4
    mask = ib == jb

    ctxs = []
    for h in range(HQ_LOC):
        q = q_all[:, h * DH:(h + 1) * DH]
        k = k_ref[0, :, h, :]
        v = v_ref[0, :, h, :]
        s = jnp.dot(q, k.T, preferred_element_type=jnp.float32) * SCALE
        s = jnp.where(mask, s, -1e9)
        m = jnp.max(s, axis=-1, keepdims=True)
        w = jnp.exp(s - m)
        w = w / jnp.sum(w, axis=-1, keepdims=True)
        ctxs.append(jnp.dot(w, v, preferred_element_type=jnp.float32))
    ctx = jnp.concatenate(ctxs, axis=1)
    partial_ref[...] = jnp.dot(ctx, wo_ref[...],
                               preferred_element_type=jnp.float32)

    barrier_sem = pltpu.get_barrier_semaphore()
    for nbr in (left, right):
        pl.semaphore_signal(barrier_sem, inc=1, device_id=(nbr,),
                            device_id_type=pl.DeviceIdType.MESH)
    pl.semaphore_wait(barrier_sem, 2)

    for s in range(N_DEV - 1):
        cs = (my + N_DEV - s) % N_DEV
        rdma = pltpu.make_async_remote_copy(
            src_ref=partial_ref.at[pl.ds(cs * CHUNK, CHUNK), :],
            dst_ref=comm_ref.at[s],
            send_sem=send_sems.at[s],
            recv_sem=recv_sems.at[s],
            device_id=(right,),
            device_id_type=pl.DeviceIdType.MESH,
        )
        rdma.start()
        rdma.wait()
        cr = (my + N_DEV - 1 - s) % N_DEV
        partial_ref[pl.ds(cr * CHUNK, CHUNK), :] = (
            partial_ref[pl.ds(cr * CHUNK, CHUNK), :] + comm_ref[s]
        )

    own = (my + 1) % N_DEV
    out_ref[0, pl.ds(own * CHUNK, CHUNK), :] = partial_ref[
        pl.ds(own * CHUNK, CHUNK), :]

    for g in range(N_DEV - 1):
        s = (N_DEV - 1) + g
        if g == 0:
            src = partial_ref.at[pl.ds(own * CHUNK, CHUNK), :]
        else:
            src = comm_ref.at[s - 1]
        rdma = pltpu.make_async_remote_copy(
            src_ref=src,
            dst_ref=comm_ref.at[s],
            send_sem=send_sems.at[s],
            recv_sem=recv_sems.at[s],
            device_id=(right,),
            device_id_type=pl.DeviceIdType.MESH,
        )
        rdma.start()
        rdma.wait()
        cr = (my + N_DEV - g) % N_DEV
        out_ref[0, pl.ds(cr * CHUNK, CHUNK), :] = comm_ref[s]


def kernel(x, Wq, K_ext, V_ext, Wo):
    i = lax.axis_index("i")
    wq_loc = lax.dynamic_slice(Wq, (0, i * DLOC), (DM, DLOC))
    wo_loc = lax.dynamic_slice(Wo, (i * DLOC, 0), (DLOC, DM))

    return pl.pallas_call(
        _body,
        out_shape=jax.ShapeDtypeStruct((1, SQ, DM), jnp.float32),
        in_specs=[pl.BlockSpec(memory_space=pltpu.VMEM)] * 5,
        out_specs=pl.BlockSpec(memory_space=pltpu.VMEM),
        scratch_shapes=[
            pltpu.VMEM((SQ, DM), jnp.float32),
            pltpu.VMEM((N_HOPS, CHUNK, DM), jnp.float32),
            pltpu.SemaphoreType.DMA((N_HOPS,)),
            pltpu.SemaphoreType.DMA((N_HOPS,)),
        ],
        compiler_params=pltpu.CompilerParams(collective_id=0),
    )(x, wq_loc, K_ext, V_ext, wo_loc)


# device time: 104836 ns/iter; 1.1106x vs baseline; 1.1106x over previous
import jax
import jax.numpy as jnp
from jax import lax
from jax.experimental import pallas as pl
from jax.experimental.pallas import tpu as pltpu

N_DEV = 4
SQ = 1024
SKV = 1024
HQ_LOC = 8
DH = 128
DM = 1024
DLOC = HQ_LOC * DH
CHUNK = SQ // N_DEV
SCALE = 0.08838834764831843
N_HOPS = 2 * (N_DEV - 1)


def _body(x_ref, wq_ref, k_ref, v_ref, wo_ref, out_ref,
          partial_ref, comm_ref, send_sems, recv_sems):
    my = lax.axis_index("i")
    left = (my + N_DEV - 1) % N_DEV
    right = (my + 1) % N_DEV

    xm = x_ref[0]
    q_all = jnp.dot(xm, wq_ref[...], preferred_element_type=jnp.float32)
    qv = q_all.reshape(4, 4, 64, DLOC)
    kv = k_ref[0].reshape(4, 4, 64, HQ_LOC, DH)
    vv = v_ref[0].reshape(4, 4, 64, HQ_LOC, DH)

    for c in range(4):
        qc = qv[:, c].reshape(CHUNK, DLOC)
        kc = kv[:, c].reshape(CHUNK, HQ_LOC, DH)
        vc = vv[:, c].reshape(CHUNK, HQ_LOC, DH)
        ctxs = []
        for h in range(HQ_LOC):
            q = qc[:, h * DH:(h + 1) * DH]
            k = kc[:, h, :]
            v = vc[:, h, :]
            s = jnp.dot(q, k.T, preferred_element_type=jnp.float32) * SCALE
            w = jnp.exp(s)
            w = w / jnp.sum(w, axis=-1, keepdims=True)
            ctxs.append(jnp.dot(w, v, preferred_element_type=jnp.float32))
        ctx_c = jnp.concatenate(ctxs, axis=1)
        p_c = jnp.dot(ctx_c, wo_ref[...], preferred_element_type=jnp.float32)
        partial_ref[:, c:c + 1, :, :] = p_c.reshape(4, 1, 64, DM)

    barrier_sem = pltpu.get_barrier_semaphore()
    for nbr in (left, right):
        pl.semaphore_signal(barrier_sem, inc=1, device_id=(nbr,),
                            device_id_type=pl.DeviceIdType.MESH)
    pl.semaphore_wait(barrier_sem, 2)

    for s in range(N_DEV - 1):
        cs = (my + N_DEV - s) % N_DEV
        rdma = pltpu.make_async_remote_copy(
            src_ref=partial_ref.at[pl.ds(cs, 1)],
            dst_ref=comm_ref.at[pl.ds(s, 1)],
            send_sem=send_sems.at[s],
            recv_sem=recv_sems.at[s],
            device_id=(right,),
            device_id_type=pl.DeviceIdType.MESH,
        )
        rdma.start()
        rdma.wait()
        cr = (my + N_DEV - 1 - s) % N_DEV
        partial_ref[pl.ds(cr, 1)] = partial_ref[pl.ds(cr, 1)] + comm_ref[s:s + 1]

    own = (my + 1) % N_DEV
    out_ref[0, pl.ds(own * CHUNK, CHUNK), :] = partial_ref[
        pl.ds(own, 1)].reshape(CHUNK, DM)

    for g in range(N_DEV - 1):
        s = (N_DEV - 1) + g
        if g == 0:
            src = partial_ref.at[pl.ds(own, 1)]
        else:
            src = comm_ref.at[pl.ds(s - 1, 1)]
        rdma = pltpu.make_async_remote_copy(
            src_ref=src,
            dst_ref=comm_ref.at[pl.ds(s, 1)],
            send_sem=send_sems.at[s],
            recv_sem=recv_sems.at[s],
            device_id=(right,),
            device_id_type=pl.DeviceIdType.MESH,
        )
        rdma.start()
        rdma.wait()
        cr = (my + N_DEV - g) % N_DEV
        out_ref[0, pl.ds(cr * CHUNK, CHUNK), :] = comm_ref[s].reshape(CHUNK, DM)


def kernel(x, Wq, K_ext, V_ext, Wo):
    i = lax.axis_index("i")
    wq_loc = lax.dynamic_slice(Wq, (0, i * DLOC), (DM, DLOC))
    wo_loc = lax.dynamic_slice(Wo, (i * DLOC, 0), (DLOC, DM))

    return pl.pallas_call(
        _body,
        out_shape=jax.ShapeDtypeStruct((1, SQ, DM), jnp.float32),
        in_specs=[pl.BlockSpec(memory_space=pltpu.VMEM)] * 5,
        out_specs=pl.BlockSpec(memory_space=pltpu.VMEM),
        scratch_shapes=[
            pltpu.VMEM((N_DEV, 4, 64, DM), jnp.float32),
            pltpu.VMEM((N_HOPS, 4, 64, DM), jnp.float32),
            pltpu.SemaphoreType.DMA((N_HOPS,)),
            pltpu.SemaphoreType.DMA((N_HOPS,)),
        ],
        compiler_params=pltpu.CompilerParams(collective_id=0),
    )(x, wq_loc, K_ext, V_ext, wo_loc)


# device time: 67889 ns/iter; 1.7150x vs baseline; 1.5442x over previous
import jax
import jax.numpy as jnp
from jax import lax
from jax.experimental import pallas as pl
from jax.experimental.pallas import tpu as pltpu

N_DEV = 4
SQ = 1024
SKV = 1024
HQ_LOC = 8
DH = 128
DM = 1024
DLOC = HQ_LOC * DH
CHUNK = SQ // N_DEV
SCALE = 0.08838834764831843
N_XCHG = 8


def _body(x_ref, wq_ref, k_ref, v_ref, wo_ref, out_ref,
          partial_ref, comm_ref, send_sems, recv_sems):
    my = lax.axis_index("i")
    left = (my + N_DEV - 1) % N_DEV
    right = (my + 1) % N_DEV

    xm = x_ref[0]
    q_all = jnp.dot(xm, wq_ref[...], preferred_element_type=jnp.float32)
    qv = q_all.reshape(4, 4, 64, DLOC)
    kv = k_ref[0].reshape(4, 4, 64, HQ_LOC, DH)
    vv = v_ref[0].reshape(4, 4, 64, HQ_LOC, DH)

    for c in range(4):
        qc = qv[:, c].reshape(CHUNK, DLOC)
        kc = kv[:, c].reshape(CHUNK, HQ_LOC, DH)
        vc = vv[:, c].reshape(CHUNK, HQ_LOC, DH)
        ctxs = []
        for h in range(HQ_LOC):
            q = qc[:, h * DH:(h + 1) * DH]
            k = kc[:, h, :]
            v = vc[:, h, :]
            s = jnp.dot(q, k.T, preferred_element_type=jnp.float32) * SCALE
            w = jnp.exp(s)
            w = w / jnp.sum(w, axis=-1, keepdims=True)
            ctxs.append(jnp.dot(w, v, preferred_element_type=jnp.float32))
        ctx_c = jnp.concatenate(ctxs, axis=1)
        p_c = jnp.dot(ctx_c, wo_ref[...], preferred_element_type=jnp.float32)
        partial_ref[:, c:c + 1, :, :] = p_c.reshape(4, 1, 64, DM)

    barrier_sem = pltpu.get_barrier_semaphore()
    for nbr in (left, right):
        pl.semaphore_signal(barrier_sem, inc=1, device_id=(nbr,),
                            device_id_type=pl.DeviceIdType.MESH)
    pl.semaphore_wait(barrier_sem, 2)

    ba = (my % 2 + my // 2) % 2
    bb = my // 2
    yp = my + 1 - 2 * (my % 2)
    xp = 3 - my

    def exch(slot, src, dst_slice, peer):
        rdma = pltpu.make_async_remote_copy(
            src_ref=src,
            dst_ref=dst_slice,
            send_sem=send_sems.at[slot],
            recv_sem=recv_sems.at[slot],
            device_id=(peer,),
            device_id_type=pl.DeviceIdType.MESH,
        )
        rdma.start()
        return rdma

    ra = exch(0, partial_ref.at[pl.ds(1 - ba, 1)], comm_ref.at[0:1], yp)
    rb = exch(1, partial_ref.at[pl.ds(3 - bb, 1)], comm_ref.at[1:2], xp)
    ra.wait()
    rb.wait()
    partial_ref[pl.ds(ba, 1)] = partial_ref[pl.ds(ba, 1)] + comm_ref[0:1]
    partial_ref[pl.ds(2 + bb, 1)] = partial_ref[pl.ds(2 + bb, 1)] + comm_ref[1:2]

    ra = exch(2, partial_ref.at[pl.ds(ba, 1), pl.ds(2 * (1 - bb), 2)],
              comm_ref.at[2:3, 0:2], xp)
    rb = exch(3, partial_ref.at[pl.ds(2 + bb, 1), pl.ds(2 * (1 - ba), 2)],
              comm_ref.at[3:4, 0:2], yp)
    ra.wait()
    rb.wait()
    partial_ref[pl.ds(ba, 1), pl.ds(2 * bb, 2)] = (
        partial_ref[pl.ds(ba, 1), pl.ds(2 * bb, 2)] + comm_ref[2:3, 0:2])
    partial_ref[pl.ds(2 + bb, 1), pl.ds(2 * ba, 2)] = (
        partial_ref[pl.ds(2 + bb, 1), pl.ds(2 * ba, 2)] + comm_ref[3:4, 0:2])

    ra = exch(4, partial_ref.at[pl.ds(ba, 1), pl.ds(2 * bb, 2)],
              comm_ref.at[4:5, 0:2], xp)
    rb = exch(5, partial_ref.at[pl.ds(2 + bb, 1), pl.ds(2 * ba, 2)],
              comm_ref.at[5:6, 0:2], yp)
    ra.wait()
    rb.wait()
    partial_ref[pl.ds(ba, 1), pl.ds(2 * (1 - bb), 2)] = comm_ref[4:5, 0:2]
    partial_ref[pl.ds(2 + bb, 1), pl.ds(2 * (1 - ba), 2)] = comm_ref[5:6, 0:2]

    ra = exch(6, partial_ref.at[pl.ds(ba, 1)], comm_ref.at[6:7], yp)
    rb = exch(7, partial_ref.at[pl.ds(2 + bb, 1)], comm_ref.at[7:8], xp)
    ra.wait()
    rb.wait()
    partial_ref[pl.ds(1 - ba, 1)] = comm_ref[6:7]
    partial_ref[pl.ds(3 - bb, 1)] = comm_ref[7:8]

    for g in range(N_DEV):
        out_ref[0, pl.ds(g * CHUNK, CHUNK), :] = partial_ref[g].reshape(
            CHUNK, DM)


def kernel(x, Wq, K_ext, V_ext, Wo):
    i = lax.axis_index("i")
    wq_loc = lax.dynamic_slice(Wq, (0, i * DLOC), (DM, DLOC))
    wo_loc = lax.dynamic_slice(Wo, (i * DLOC, 0), (DLOC, DM))

    return pl.pallas_call(
        _body,
        out_shape=jax.ShapeDtypeStruct((1, SQ, DM), jnp.float32),
        in_specs=[pl.BlockSpec(memory_space=pltpu.VMEM)] * 5,
        out_specs=pl.BlockSpec(memory_space=pltpu.VMEM),
        scratch_shapes=[
            pltpu.VMEM((N_DEV, 4, 64, DM), jnp.float32),
            pltpu.VMEM((N_XCHG, 4, 64, DM), jnp.float32),
            pltpu.SemaphoreType.DMA((N_XCHG,)),
            pltpu.SemaphoreType.DMA((N_XCHG,)),
        ],
        compiler_params=pltpu.CompilerParams(collective_id=0),
    )(x, wq_loc, K_ext, V_ext, wo_loc)


# device time: 63855 ns/iter; 1.8234x vs baseline; 1.0632x over previous
import jax
import jax.numpy as jnp
from jax import lax
from jax.experimental import pallas as pl
from jax.experimental.pallas import tpu as pltpu

N_DEV = 4
SQ = 1024
SKV = 1024
HQ_LOC = 8
DH = 128
DM = 1024
DLOC = HQ_LOC * DH
CHUNK = SQ // N_DEV
HALF = CHUNK // 2
SCALE = 0.08838834764831843
N_XCHG = 8


def _body(x_ref, wq_ref, k_ref, v_ref, wo_ref, out_ref,
          xs_ref, wqs_ref, ks_ref, vs_ref, wos_ref,
          ctx_ref, partial_ref, comm_ref,
          cp_sems, send_sems, recv_sems):
    my = lax.axis_index("i")

    cp_x = pltpu.make_async_copy(x_ref.at[0], xs_ref, cp_sems.at[0])
    cp_wq = pltpu.make_async_copy(
        wq_ref.at[:, pl.ds(my * DLOC, DLOC)], wqs_ref, cp_sems.at[1])
    cp_k = pltpu.make_async_copy(k_ref.at[0], ks_ref, cp_sems.at[2])
    cp_v = pltpu.make_async_copy(v_ref.at[0], vs_ref, cp_sems.at[3])
    cp_wo = pltpu.make_async_copy(
        wo_ref.at[pl.ds(my * DLOC, DLOC), :], wos_ref, cp_sems.at[4])
    for cp in (cp_x, cp_wq, cp_k, cp_v, cp_wo):
        cp.start()
    cp_x.wait()
    cp_wq.wait()

    q_all = jnp.dot(xs_ref[...], wqs_ref[...],
                    preferred_element_type=jnp.float32)
    qv = q_all.reshape(4, 4, 64, DLOC)
    cp_k.wait()
    cp_v.wait()
    kv = ks_ref[...].reshape(4, 4, 64, HQ_LOC, DH)
    vv = vs_ref[...].reshape(4, 4, 64, HQ_LOC, DH)

    for c in range(4):
        qc = qv[:, c].reshape(CHUNK, DLOC)
        kc = kv[:, c].reshape(CHUNK, HQ_LOC, DH)
        vc = vv[:, c].reshape(CHUNK, HQ_LOC, DH)
        ctxs = []
        for h in range(HQ_LOC):
            q = qc[:, h * DH:(h + 1) * DH]
            k = kc[:, h, :]
            v = vc[:, h, :]
            s = jnp.dot(q, k.T, preferred_element_type=jnp.float32) * SCALE
            w = jnp.exp(s)
            w = w / jnp.sum(w, axis=-1, keepdims=True)
            ctxs.append(jnp.dot(w, v, preferred_element_type=jnp.float32))
        ctx_c = jnp.concatenate(ctxs, axis=1)
        ctx_ref[:, c:c + 1] = ctx_c.reshape(4, 1, 64, DLOC)

    ba = (my % 2 + my // 2) % 2
    bb = my // 2
    yp = my + 1 - 2 * (my % 2)
    xp = 3 - my

    def exch(slot, src, dst_slice, peer):
        rdma = pltpu.make_async_remote_copy(
            src_ref=src,
            dst_ref=dst_slice,
            send_sem=send_sems.at[slot],
            recv_sem=recv_sems.at[slot],
            device_id=(peer,),
            device_id_type=pl.DeviceIdType.MESH,
        )
        rdma.start()
        return rdma

    cp_wo.wait()
    wos = wos_ref[...]

    barrier_sem = pltpu.get_barrier_semaphore()
    ra = rb = None
    for step, g in enumerate([1 - ba, 3 - bb, ba, 2 + bb]):
        ctg = ctx_ref[pl.ds(g, 1)].reshape(CHUNK, DLOC)
        p = jnp.dot(ctg, wos, preferred_element_type=jnp.float32)
        partial_ref[pl.ds(g, 1)] = p.reshape(1, CHUNK, DM)
        if step == 1:
            for nbr in (yp, xp):
                pl.semaphore_signal(barrier_sem, inc=1, device_id=(nbr,),
                                    device_id_type=pl.DeviceIdType.MESH)
            pl.semaphore_wait(barrier_sem, 2)
            ra = exch(0, partial_ref.at[pl.ds(1 - ba, 1)], comm_ref.at[0:1], yp)
            rb = exch(1, partial_ref.at[pl.ds(3 - bb, 1)], comm_ref.at[1:2], xp)
    ra.wait()
    rb.wait()
    partial_ref[pl.ds(ba, 1)] = partial_ref[pl.ds(ba, 1)] + comm_ref[0:1]
    partial_ref[pl.ds(2 + bb, 1)] = (
        partial_ref[pl.ds(2 + bb, 1)] + comm_ref[1:2])

    ra = exch(2, partial_ref.at[pl.ds(ba, 1), pl.ds(HALF * (1 - bb), HALF)],
              comm_ref.at[2:3, 0:HALF], xp)
    rb = exch(3, partial_ref.at[pl.ds(2 + bb, 1), pl.ds(HALF * (1 - ba), HALF)],
              comm_ref.at[3:4, 0:HALF], yp)
    ra.wait()
    rb.wait()
    partial_ref[pl.ds(ba, 1), pl.ds(HALF * bb, HALF)] = (
        partial_ref[pl.ds(ba, 1), pl.ds(HALF * bb, HALF)]
        + comm_ref[2:3, 0:HALF])
    partial_ref[pl.ds(2 + bb, 1), pl.ds(HALF * ba, HALF)] = (
        partial_ref[pl.ds(2 + bb, 1), pl.ds(HALF * ba, HALF)]
        + comm_ref[3:4, 0:HALF])

    ra = exch(4, partial_ref.at[pl.ds(ba, 1), pl.ds(HALF * bb, HALF)],
              comm_ref.at[4:5, 0:HALF], xp)
    rb = exch(5, partial_ref.at[pl.ds(2 + bb, 1), pl.ds(HALF * ba, HALF)],
              comm_ref.at[5:6, 0:HALF], yp)
    ra.wait()
    rb.wait()
    partial_ref[pl.ds(ba, 1), pl.ds(HALF * (1 - bb), HALF)] = (
        comm_ref[4:5, 0:HALF])
    partial_ref[pl.ds(2 + bb, 1), pl.ds(HALF * (1 - ba), HALF)] = (
        comm_ref[5:6, 0:HALF])

    ra = exch(6, partial_ref.at[pl.ds(ba, 1)],
              out_ref.at[0:1, pl.ds(ba * CHUNK, CHUNK), :], yp)
    rb = exch(7, partial_ref.at[pl.ds(2 + bb, 1)],
              out_ref.at[0:1, pl.ds((2 + bb) * CHUNK, CHUNK), :], xp)
    out_ref[0, pl.ds(ba * CHUNK, CHUNK), :] = partial_ref[
        pl.ds(ba, 1)].reshape(CHUNK, DM)
    out_ref[0, pl.ds((2 + bb) * CHUNK, CHUNK), :] = partial_ref[
        pl.ds(2 + bb, 1)].reshape(CHUNK, DM)
    ra.wait()
    rb.wait()


def kernel(x, Wq, K_ext, V_ext, Wo):
    return pl.pallas_call(
        _body,
        out_shape=jax.ShapeDtypeStruct((1, SQ, DM), jnp.float32),
        in_specs=[pl.BlockSpec(memory_space=pl.ANY)] * 5,
        out_specs=pl.BlockSpec(memory_space=pltpu.VMEM),
        scratch_shapes=[
            pltpu.VMEM((SQ, DM), jnp.float32),
            pltpu.VMEM((DM, DLOC), jnp.float32),
            pltpu.VMEM((SKV, HQ_LOC, DH), jnp.float32),
            pltpu.VMEM((SKV, HQ_LOC, DH), jnp.float32),
            pltpu.VMEM((DLOC, DM), jnp.float32),
            pltpu.VMEM((4, 4, 64, DLOC), jnp.float32),
            pltpu.VMEM((N_DEV, CHUNK, DM), jnp.float32),
            pltpu.VMEM((N_XCHG, CHUNK, DM), jnp.float32),
            pltpu.SemaphoreType.DMA((5,)),
            pltpu.SemaphoreType.DMA((N_XCHG,)),
            pltpu.SemaphoreType.DMA((N_XCHG,)),
        ],
        compiler_params=pltpu.CompilerParams(
            collective_id=0, vmem_limit_bytes=100 * 1024 * 1024),
    )(x, Wq, K_ext, V_ext, Wo)


# device time: 47033 ns/iter; 2.4755x vs baseline; 1.3577x over previous
import jax
import jax.numpy as jnp
from jax import lax
from jax.experimental import pallas as pl
from jax.experimental.pallas import tpu as pltpu

N_DEV = 4
SQ = 1024
SKV = 1024
HQ_LOC = 8
DH = 128
DM = 1024
DLOC = HQ_LOC * DH
CHUNK = SQ // N_DEV
HALF = CHUNK // 2
SCALE = 0.08838834764831843
N_XCHG = 8


def _body(x_ref, wq_ref, k_ref, v_ref, wo_ref, out_ref,
          xs_ref, wqs_ref, ks_ref, vs_ref, wos_ref,
          ctx_ref, partial_ref, comm_ref,
          cp_sems, send_sems, recv_sems):
    my = lax.axis_index("i")

    cp_x = pltpu.make_async_copy(x_ref.at[0], xs_ref, cp_sems.at[0])
    cp_wq = pltpu.make_async_copy(
        wq_ref.at[:, pl.ds(my * DLOC, DLOC)], wqs_ref, cp_sems.at[1])
    cp_k = pltpu.make_async_copy(k_ref.at[0], ks_ref, cp_sems.at[2])
    cp_v = pltpu.make_async_copy(v_ref.at[0], vs_ref, cp_sems.at[3])
    cp_wo = pltpu.make_async_copy(
        wo_ref.at[pl.ds(my * DLOC, DLOC), :], wos_ref, cp_sems.at[4])
    for cp in (cp_x, cp_wq, cp_k, cp_v, cp_wo):
        cp.start()
    cp_x.wait()
    cp_wq.wait()

    q_all = jnp.dot(xs_ref[...].astype(jnp.bfloat16),
                    wqs_ref[...].astype(jnp.bfloat16),
                    preferred_element_type=jnp.float32)
    qv = q_all.reshape(4, 4, 64, DLOC)
    cp_k.wait()
    cp_v.wait()
    kv = ks_ref[...].reshape(4, 4, 64, HQ_LOC, DH)
    vv = vs_ref[...].reshape(4, 4, 64, HQ_LOC, DH)

    for c in range(4):
        qc = qv[:, c].reshape(CHUNK, DLOC)
        kc = kv[:, c].reshape(CHUNK, HQ_LOC, DH)
        vc = vv[:, c].reshape(CHUNK, HQ_LOC, DH)
        ctxs = []
        for h in range(HQ_LOC):
            q = qc[:, h * DH:(h + 1) * DH]
            k = kc[:, h, :]
            v = vc[:, h, :]
            s = jnp.dot(q, k.T, preferred_element_type=jnp.float32) * SCALE
            w = jnp.exp(s)
            w = w / jnp.sum(w, axis=-1, keepdims=True)
            ctxs.append(jnp.dot(w, v, preferred_element_type=jnp.float32))
        ctx_c = jnp.concatenate(ctxs, axis=1)
        ctx_ref[:, c:c + 1] = ctx_c.astype(jnp.bfloat16).reshape(4, 1, 64, DLOC)

    ba = (my % 2 + my // 2) % 2
    bb = my // 2
    yp = my + 1 - 2 * (my % 2)
    xp = 3 - my

    def exch(slot, src, dst_slice, peer):
        rdma = pltpu.make_async_remote_copy(
            src_ref=src,
            dst_ref=dst_slice,
            send_sem=send_sems.at[slot],
            recv_sem=recv_sems.at[slot],
            device_id=(peer,),
            device_id_type=pl.DeviceIdType.MESH,
        )
        rdma.start()
        return rdma

    cp_wo.wait()
    wos = wos_ref[...].astype(jnp.bfloat16)

    barrier_sem = pltpu.get_barrier_semaphore()
    ra = rb = None
    for step, g in enumerate([1 - ba, 3 - bb, ba, 2 + bb]):
        ctg = ctx_ref[pl.ds(g, 1)].reshape(CHUNK, DLOC)
        p = jnp.dot(ctg, wos, preferred_element_type=jnp.float32)
        partial_ref[pl.ds(g, 1)] = p.astype(jnp.bfloat16).reshape(1, CHUNK, DM)
        if step == 1:
            for nbr in (yp, xp):
                pl.semaphore_signal(barrier_sem, inc=1, device_id=(nbr,),
                                    device_id_type=pl.DeviceIdType.MESH)
            pl.semaphore_wait(barrier_sem, 2)
            ra = exch(0, partial_ref.at[pl.ds(1 - ba, 1)], comm_ref.at[0:1], yp)
            rb = exch(1, partial_ref.at[pl.ds(3 - bb, 1)], comm_ref.at[1:2], xp)
    ra.wait()
    rb.wait()
    partial_ref[pl.ds(ba, 1)] = partial_ref[pl.ds(ba, 1)] + comm_ref[0:1]
    partial_ref[pl.ds(2 + bb, 1)] = (
        partial_ref[pl.ds(2 + bb, 1)] + comm_ref[1:2])

    ra = exch(2, partial_ref.at[pl.ds(ba, 1), pl.ds(HALF * (1 - bb), HALF)],
              comm_ref.at[2:3, 0:HALF], xp)
    rb = exch(3, partial_ref.at[pl.ds(2 + bb, 1), pl.ds(HALF * (1 - ba), HALF)],
              comm_ref.at[3:4, 0:HALF], yp)
    ra.wait()
    rb.wait()
    partial_ref[pl.ds(ba, 1), pl.ds(HALF * bb, HALF)] = (
        partial_ref[pl.ds(ba, 1), pl.ds(HALF * bb, HALF)]
        + comm_ref[2:3, 0:HALF])
    partial_ref[pl.ds(2 + bb, 1), pl.ds(HALF * ba, HALF)] = (
        partial_ref[pl.ds(2 + bb, 1), pl.ds(HALF * ba, HALF)]
        + comm_ref[3:4, 0:HALF])

    ra = exch(4, partial_ref.at[pl.ds(ba, 1), pl.ds(HALF * bb, HALF)],
              comm_ref.at[4:5, 0:HALF], xp)
    rb = exch(5, partial_ref.at[pl.ds(2 + bb, 1), pl.ds(HALF * ba, HALF)],
              comm_ref.at[5:6, 0:HALF], yp)
    ra.wait()
    rb.wait()
    partial_ref[pl.ds(ba, 1), pl.ds(HALF * (1 - bb), HALF)] = (
        comm_ref[4:5, 0:HALF])
    partial_ref[pl.ds(2 + bb, 1), pl.ds(HALF * (1 - ba), HALF)] = (
        comm_ref[5:6, 0:HALF])

    ra = exch(6, partial_ref.at[pl.ds(ba, 1)], comm_ref.at[6:7], yp)
    rb = exch(7, partial_ref.at[pl.ds(2 + bb, 1)], comm_ref.at[7:8], xp)
    out_ref[0, pl.ds(ba * CHUNK, CHUNK), :] = partial_ref[
        pl.ds(ba, 1)].astype(jnp.float32).reshape(CHUNK, DM)
    out_ref[0, pl.ds((2 + bb) * CHUNK, CHUNK), :] = partial_ref[
        pl.ds(2 + bb, 1)].astype(jnp.float32).reshape(CHUNK, DM)
    ra.wait()
    rb.wait()
    out_ref[0, pl.ds((1 - ba) * CHUNK, CHUNK), :] = comm_ref[
        6:7].astype(jnp.float32).reshape(CHUNK, DM)
    out_ref[0, pl.ds((3 - bb) * CHUNK, CHUNK), :] = comm_ref[
        7:8].astype(jnp.float32).reshape(CHUNK, DM)


def kernel(x, Wq, K_ext, V_ext, Wo):
    return pl.pallas_call(
        _body,
        out_shape=jax.ShapeDtypeStruct((1, SQ, DM), jnp.float32),
        in_specs=[pl.BlockSpec(memory_space=pl.ANY)] * 5,
        out_specs=pl.BlockSpec(memory_space=pltpu.VMEM),
        scratch_shapes=[
            pltpu.VMEM((SQ, DM), jnp.float32),
            pltpu.VMEM((DM, DLOC), jnp.float32),
            pltpu.VMEM((SKV, HQ_LOC, DH), jnp.float32),
            pltpu.VMEM((SKV, HQ_LOC, DH), jnp.float32),
            pltpu.VMEM((DLOC, DM), jnp.float32),
            pltpu.VMEM((4, 4, 64, DLOC), jnp.bfloat16),
            pltpu.VMEM((N_DEV, CHUNK, DM), jnp.bfloat16),
            pltpu.VMEM((N_XCHG, CHUNK, DM), jnp.bfloat16),
            pltpu.SemaphoreType.DMA((5,)),
            pltpu.SemaphoreType.DMA((N_XCHG,)),
            pltpu.SemaphoreType.DMA((N_XCHG,)),
        ],
        compiler_params=pltpu.CompilerParams(
            collective_id=0, vmem_limit_bytes=100 * 1024 * 1024),
    )(x, Wq, K_ext, V_ext, Wo)
